# baseline (speedup 1.0000x reference)
import functools

import jax
import jax.numpy as jnp
from jax import lax
from jax.experimental import pallas as pl
from jax.experimental.pallas import tpu as pltpu
from jax.experimental.pallas import tpu_sc as plsc

_NC = 2
_NS = 16
_NW = _NC * _NS
_D = 64
_L = 16
_IB = 128
_KB = 2
_CHUNK = _KB * _IB

_S = 200
_B = 4096
_BH = _B // _IB
_FH = _D // 8


@functools.cache
def _make_embed():
    n_blocks = _S * _BH
    blocks_w = n_blocks // _NW
    n_pairs = blocks_w // (2 * _KB)
    mesh = plsc.VectorSubcoreMesh(core_axis_name="c", subcore_axis_name="s")

    @functools.partial(
        pl.kernel,
        out_type=jax.ShapeDtypeStruct((_S, _FH, _BH, 8, _IB), jnp.float32),
        mesh=mesh,
        scratch_types=[
            pltpu.VMEM((_S * _BH // _NW, _IB), jnp.int32),
            pltpu.VMEM((_CHUNK, _D), jnp.float32),
            pltpu.VMEM((_CHUNK, _D), jnp.float32),
            pltpu.VMEM((_CHUNK * (_D + 1),), jnp.float32),
            pltpu.VMEM((_FH, _KB, 8, _IB), jnp.float32),
            pltpu.VMEM((_FH, _KB, 8, _IB), jnp.float32),
            pltpu.SemaphoreType.DMA,
            pltpu.SemaphoreType.DMA,
            pltpu.SemaphoreType.DMA,
            pltpu.SemaphoreType.DMA,
        ],
        compiler_params=pltpu.CompilerParams(
            use_tc_tiling_on_sc=False, needs_layout_passes=False,
            disable_bounds_checks=True),
    )
    def k(xt_hbm, table_hbm, out_hbm,
          idx_all, raw0, raw1, raw65, tr0, tr1, g0, g1, o0, o1):
        raw = (raw0, raw1)
        tr = (tr0, tr1)
        gsem = (g0, g1)
        osem = (o0, o1)
        wid = lax.axis_index("s") * _NC + lax.axis_index("c")
        base_blk = wid * blocks_w
        iota = lax.iota(jnp.int32, _L)

        def drain_out(b):
            pltpu.make_async_copy(
                tr[b], out_hbm.at[0, :, pl.ds(0, _KB)], osem[b]).wait()

        def drain_gather(b):
            pltpu.make_async_copy(
                table_hbm.at[pl.ds(0, _CHUNK)], raw[b], gsem[b]).wait()

        iota65 = iota * (_D + 1)

        def transpose_chunk(b):
            def cbody(t, carry):
                vals = []
                for u in range(4):
                    tt = t * 4 + u
                    for g in range(_D // _L):
                        vals.append((tt, g, raw[b][tt, pl.ds(g * _L, _L)]))
                for tt, g, v in vals:
                    raw65[pl.ds(tt * (_D + 1) + g * _L, _L)] = v
                return carry

            lax.fori_loop(0, _CHUNK // 4, cbody, 0)

            for tg in range(_IB // _L):
                tg65 = tg * (_L * (_D + 1))
                pairs = [(f, kk, iota65 + (tg65 + (kk * _IB * (_D + 1) + f)))
                         for f in range(_D) for kk in range(_KB)]
                for gi in range(0, len(pairs), 16):
                    grp = pairs[gi:gi + 16]
                    vals = [plsc.load_gather(raw65, [iv]) for _, _, iv in grp]
                    for (f, kk, _), v in zip(grp, vals):
                        tr[b][f // 8, kk, f % 8, pl.ds(tg * _L, _L)] = v

        def fire_gathers(c, b):
            for j in range(_KB):
                pltpu.async_copy(
                    table_hbm.at[idx_all.at[c * _KB + j]],
                    raw[b].at[pl.ds(j * _IB, _IB)], gsem[b])

        def fire_out(c, b):
            n0 = base_blk + c * _KB
            s, bh0 = n0 // _BH, n0 % _BH
            for fh in range(_FH):
                pltpu.async_copy(
                    tr[b].at[fh],
                    out_hbm.at[s, fh, pl.ds(bh0, _KB)], osem[b])

        def body(i, carry):
            c0 = 2 * i
            fire_gathers(c0 + 1, 1)
            drain_gather(0)

            @pl.when(i > 0)
            def _():
                drain_out(0)

            transpose_chunk(0)
            fire_out(c0, 0)

            @pl.when(i + 1 < n_pairs)
            def _():
                fire_gathers(c0 + 2, 0)

            drain_gather(1)

            @pl.when(i > 0)
            def _():
                drain_out(1)

            transpose_chunk(1)
            fire_out(c0 + 1, 1)
            return carry

        pltpu.sync_copy(xt_hbm.at[pl.ds(base_blk, blocks_w)], idx_all)
        fire_gathers(0, 0)
        lax.fori_loop(0, n_pairs, body, 0)
        for b in range(2):
            drain_out(b)

    return k


def kernel(x, table):
    assert x.shape == (_B, _S) and table.shape[1] == _D
    xt = jnp.transpose(x).reshape(_S * _BH, _IB)
    o5 = _make_embed()(xt, table)
    return o5.transpose(2, 4, 0, 1, 3).reshape(_B, _S, _D)


# speedup vs baseline: 1.3439x; 1.3439x over previous
import functools

import jax
import jax.numpy as jnp
from jax import lax
from jax.experimental import pallas as pl
from jax.experimental.pallas import tpu as pltpu
from jax.experimental.pallas import tpu_sc as plsc

_NC = 2
_NS = 16
_NW = _NC * _NS
_D = 64
_L = 16
_IB = 128
_KB = 2
_CHUNK = _KB * _IB

_S = 200
_B = 4096
_BH = _B // _IB
_FH = _D // 8


@functools.cache
def _make_embed():
    n_blocks = _S * _BH
    blocks_w = n_blocks // _NW
    n_pairs = blocks_w // (2 * _KB)
    mesh = plsc.VectorSubcoreMesh(core_axis_name="c", subcore_axis_name="s")

    @functools.partial(
        pl.kernel,
        out_type=jax.ShapeDtypeStruct((_S, _FH, _BH, 8, _IB), jnp.float32),
        mesh=mesh,
        scratch_types=[
            pltpu.VMEM((_S * _BH // _NW, _IB), jnp.int32),
            pltpu.VMEM((_CHUNK, _D), jnp.float32),
            pltpu.VMEM((_CHUNK, _D), jnp.float32),
            pltpu.VMEM((_CHUNK * (_D + 1),), jnp.float32),
            pltpu.VMEM((_FH, _KB, 8, _IB), jnp.float32),
            pltpu.VMEM((_FH, _KB, 8, _IB), jnp.float32),
            pltpu.SemaphoreType.DMA,
            pltpu.SemaphoreType.DMA,
            pltpu.SemaphoreType.DMA,
            pltpu.SemaphoreType.DMA,
        ],
        compiler_params=pltpu.CompilerParams(
            use_tc_tiling_on_sc=False, needs_layout_passes=False,
            disable_bounds_checks=True),
    )
    def k(xt_hbm, table_hbm, out_hbm,
          idx_all, raw0, raw1, raw65, tr0, tr1, g0, g1, o0, o1):
        raw = (raw0, raw1)
        tr = (tr0, tr1)
        gsem = (g0, g1)
        osem = (o0, o1)
        wid = lax.axis_index("s") * _NC + lax.axis_index("c")
        base_blk = wid * blocks_w
        iota = lax.iota(jnp.int32, _L)

        def drain_out(b):
            pltpu.make_async_copy(
                tr[b], out_hbm.at[0, :, pl.ds(0, _KB)], osem[b]).wait()

        def drain_gather(b):
            pltpu.make_async_copy(
                table_hbm.at[pl.ds(0, _CHUNK)], raw[b], gsem[b]).wait()

        iota65 = iota * (_D + 1)

        def transpose_chunk(b):
            def cbody(t, carry):
                vals = []
                for u in range(4):
                    tt = t * 4 + u
                    for g in range(_D // _L):
                        vals.append((tt, g, raw[b][tt, pl.ds(g * _L, _L)]))
                for tt, g, v in vals:
                    raw65[pl.ds(tt * (_D + 1) + g * _L, _L)] = v
                return carry

            lax.fori_loop(0, _CHUNK // 4, cbody, 0)

            def tbody(tg, carry):
                tg65 = tg * (_L * (_D + 1))
                pairs = [(f, kk, iota65 + (tg65 + (kk * _IB * (_D + 1) + f)))
                         for f in range(_D) for kk in range(_KB)]
                for gi in range(0, len(pairs), 16):
                    grp = pairs[gi:gi + 16]
                    vals = [plsc.load_gather(raw65, [iv]) for _, _, iv in grp]
                    for (f, kk, _), v in zip(grp, vals):
                        tr[b][f // 8, kk, f % 8, pl.ds(tg * _L, _L)] = v
                return carry

            lax.fori_loop(0, _IB // _L, tbody, 0)

        def fire_gathers(c, b):
            for j in range(_KB):
                pltpu.async_copy(
                    table_hbm.at[idx_all.at[c * _KB + j]],
                    raw[b].at[pl.ds(j * _IB, _IB)], gsem[b])

        def fire_out(c, b):
            n0 = base_blk + c * _KB
            s, bh0 = n0 // _BH, n0 % _BH
            for fh in range(_FH):
                pltpu.async_copy(
                    tr[b].at[fh],
                    out_hbm.at[s, fh, pl.ds(bh0, _KB)], osem[b])

        def body(i, carry):
            c0 = 2 * i
            fire_gathers(c0 + 1, 1)
            drain_gather(0)

            @pl.when(i > 0)
            def _():
                drain_out(0)

            transpose_chunk(0)
            fire_out(c0, 0)

            @pl.when(i + 1 < n_pairs)
            def _():
                fire_gathers(c0 + 2, 0)

            drain_gather(1)

            @pl.when(i > 0)
            def _():
                drain_out(1)

            transpose_chunk(1)
            fire_out(c0 + 1, 1)
            return carry

        pltpu.sync_copy(xt_hbm.at[pl.ds(base_blk, blocks_w)], idx_all)
        fire_gathers(0, 0)
        lax.fori_loop(0, n_pairs, body, 0)
        for b in range(2):
            drain_out(b)

    return k


def kernel(x, table):
    assert x.shape == (_B, _S) and table.shape[1] == _D
    xt = jnp.transpose(x).reshape(_S * _BH, _IB)
    o5 = _make_embed()(xt, table)
    return o5.transpose(2, 4, 0, 1, 3).reshape(_B, _S, _D)


# speedup vs baseline: 1.3457x; 1.0013x over previous
import functools

import jax
import jax.numpy as jnp
from jax import lax
from jax.experimental import pallas as pl
from jax.experimental.pallas import tpu as pltpu
from jax.experimental.pallas import tpu_sc as plsc

_NC = 2
_NS = 16
_NW = _NC * _NS
_D = 64
_L = 16
_IB = 128
_KB = 2
_CHUNK = _KB * _IB

_S = 200
_B = 4096
_BH = _B // _IB
_FH = _D // 8


@functools.cache
def _make_embed():
    n_blocks = _S * _BH
    blocks_w = n_blocks // _NW
    n_pairs = blocks_w // (2 * _KB)
    mesh = plsc.VectorSubcoreMesh(core_axis_name="c", subcore_axis_name="s")

    @functools.partial(
        pl.kernel,
        out_type=jax.ShapeDtypeStruct((_S, _FH, _BH, 8, _IB), jnp.float32),
        mesh=mesh,
        scratch_types=[
            pltpu.VMEM((_S * _BH // _NW, _IB), jnp.int32),
            pltpu.VMEM((_CHUNK, _D), jnp.float32),
            pltpu.VMEM((_CHUNK, _D), jnp.float32),
            pltpu.VMEM((_CHUNK * (_D + 1),), jnp.float32),
            pltpu.VMEM((_FH, _KB, 8, _IB), jnp.float32),
            pltpu.VMEM((_FH, _KB, 8, _IB), jnp.float32),
            pltpu.SemaphoreType.DMA,
            pltpu.SemaphoreType.DMA,
            pltpu.SemaphoreType.DMA,
            pltpu.SemaphoreType.DMA,
        ],
        compiler_params=pltpu.CompilerParams(
            use_tc_tiling_on_sc=False, needs_layout_passes=False,
            disable_bounds_checks=True),
    )
    def k(xt_hbm, table_hbm, out_hbm,
          idx_all, raw0, raw1, raw65, tr0, tr1, g0, g1, o0, o1):
        raw = (raw0, raw1)
        tr = (tr0, tr1)
        gsem = (g0, g1)
        osem = (o0, o1)
        wid = lax.axis_index("s") * _NC + lax.axis_index("c")
        base_blk = wid * blocks_w
        iota = lax.iota(jnp.int32, _L)

        def drain_out(b):
            pltpu.make_async_copy(
                tr[b], out_hbm.at[0, :, pl.ds(0, _KB)], osem[b]).wait()

        def drain_gather(b):
            pltpu.make_async_copy(
                table_hbm.at[pl.ds(0, _CHUNK)], raw[b], gsem[b]).wait()

        iota65 = iota * (_D + 1)

        def transpose_chunk(b):
            def cbody(t, carry):
                for h in range(2):
                    vals = []
                    for u in range(4):
                        tt = t * 8 + h * 4 + u
                        for g in range(_D // _L):
                            vals.append(
                                (tt, g, raw[b][tt, pl.ds(g * _L, _L)]))
                    for tt, g, v in vals:
                        raw65[pl.ds(tt * (_D + 1) + g * _L, _L)] = v
                return carry

            lax.fori_loop(0, _CHUNK // 8, cbody, 0)

            def tbody(tg, carry):
                tg65 = tg * (_L * (_D + 1))
                pairs = [(f, kk, iota65 + (tg65 + (kk * _IB * (_D + 1) + f)))
                         for f in range(_D) for kk in range(_KB)]
                for gi in range(0, len(pairs), 16):
                    grp = pairs[gi:gi + 16]
                    vals = [plsc.load_gather(raw65, [iv]) for _, _, iv in grp]
                    for (f, kk, _), v in zip(grp, vals):
                        tr[b][f // 8, kk, f % 8, pl.ds(tg * _L, _L)] = v
                return carry

            lax.fori_loop(0, _IB // _L, tbody, 0)

        def fire_gathers(c, b):
            for j in range(_KB):
                pltpu.async_copy(
                    table_hbm.at[idx_all.at[c * _KB + j]],
                    raw[b].at[pl.ds(j * _IB, _IB)], gsem[b])

        def fire_out(c, b):
            n0 = base_blk + c * _KB
            s, bh0 = n0 // _BH, n0 % _BH
            for fh in range(_FH):
                pltpu.async_copy(
                    tr[b].at[fh],
                    out_hbm.at[s, fh, pl.ds(bh0, _KB)], osem[b])

        def body(i, carry):
            c0 = 2 * i
            fire_gathers(c0 + 1, 1)
            drain_gather(0)

            @pl.when(i > 0)
            def _():
                drain_out(0)

            transpose_chunk(0)
            fire_out(c0, 0)

            @pl.when(i + 1 < n_pairs)
            def _():
                fire_gathers(c0 + 2, 0)

            drain_gather(1)

            @pl.when(i > 0)
            def _():
                drain_out(1)

            transpose_chunk(1)
            fire_out(c0 + 1, 1)
            return carry

        pltpu.sync_copy(xt_hbm.at[pl.ds(base_blk, blocks_w)], idx_all)
        fire_gathers(0, 0)
        lax.fori_loop(0, n_pairs, body, 0)
        for b in range(2):
            drain_out(b)

    return k


def kernel(x, table):
    assert x.shape == (_B, _S) and table.shape[1] == _D
    xt = jnp.transpose(x).reshape(_S * _BH, _IB)
    o5 = _make_embed()(xt, table)
    return o5.transpose(2, 4, 0, 1, 3).reshape(_B, _S, _D)
